# baseline (speedup 1.0000x reference)
import functools

import jax
import jax.numpy as jnp
from jax import lax
from jax.experimental import pallas as pl
from jax.experimental.pallas import tpu as pltpu
from jax.experimental.pallas import tpu_sc as plsc

_B = 16384
_V = 1000000
_D = 64
_NC = 2
_NS = 16
_NW = _NC * _NS
_RPW = _B // _NW
_G = 16
_NG = _RPW // _G
_RING = 2


def _make_sc_kernel():
    mesh = plsc.VectorSubcoreMesh(core_axis_name="c", subcore_axis_name="s")

    @functools.partial(
        pl.kernel,
        out_type=jax.ShapeDtypeStruct((_B,), jnp.float32),
        mesh=mesh,
        scratch_types=[
            pltpu.VMEM((_RPW,), jnp.int32),
            pltpu.VMEM((_RPW,), jnp.int32),
            [pltpu.VMEM((_D, 128), jnp.float32) for _ in range(_RING)],
            [pltpu.VMEM((_D, 128), jnp.float32) for _ in range(_RING)],
            pltpu.VMEM((_RPW,), jnp.float32),
            pltpu.SemaphoreType.DMA,
            pltpu.SemaphoreType.DMA,
        ],
        compiler_params=pltpu.CompilerParams(needs_layout_passes=False),
    )
    def sc_kernel(ctx_hbm, word_hbm, ctx_tab_t, word_tab_t, out_hbm,
                  ctx_idx, word_idx, rc, rw, acc, sem_c, sem_w):
        wid = lax.axis_index("s") * _NC + lax.axis_index("c")
        base = wid * _RPW

        pltpu.sync_copy(ctx_hbm.at[pl.ds(base, _RPW)], ctx_idx)
        pltpu.sync_copy(word_hbm.at[pl.ds(base, _RPW)], word_idx)

        lane = lax.iota(jnp.int32, 16)
        dvec = [lane + 16 * m for m in range(_D // 16)]

        def block_dma(tab, col_blk, dst, sem):
            start = pl.multiple_of(col_blk * 128, 128)
            pltpu.async_copy(tab.at[:, pl.ds(start, 128)], dst, sem)

        def drain_one(tab, dst, sem):
            pltpu.make_async_copy(tab.at[:, pl.ds(0, 128)], dst, sem).wait()

        def extract_dot(cbuf, wbuf, c_lane, w_lane):
            c_col = jnp.full((16,), c_lane, jnp.int32)
            w_col = jnp.full((16,), w_lane, jnp.int32)
            s = jnp.zeros((16,), jnp.float32)
            for m in range(_D // 16):
                a = plsc.load_gather(cbuf, [dvec[m], c_col])
                b = plsc.load_gather(wbuf, [dvec[m], w_col])
                s = s + a * b
            return jnp.sum(s)

        icv0 = ctx_idx[pl.ds(0, _G)]
        iwv0 = word_idx[pl.ds(0, _G)]
        for k in range(_RING):
            block_dma(ctx_tab_t, icv0[k] >> 7, rc[k], sem_c)
            block_dma(word_tab_t, iwv0[k] >> 7, rw[k], sem_w)

        def body(g, _):
            icv = ctx_idx[pl.ds(g * _G, _G)]
            iwv = word_idx[pl.ds(g * _G, _G)]
            nxt = jnp.minimum(g + 1, _NG - 1)
            icv_n = ctx_idx[pl.ds(nxt * _G, _G)]
            iwv_n = word_idx[pl.ds(nxt * _G, _G)]
            res = jnp.zeros((16,), jnp.float32)
            for k in range(_G):
                slot = k % _RING
                drain_one(ctx_tab_t, rc[slot], sem_c)
                drain_one(word_tab_t, rw[slot], sem_w)
                d = extract_dot(rc[slot], rw[slot],
                                icv[k] & 127, iwv[k] & 127)
                res = jnp.where(lane == k, d, res)
                if k + _RING < _G:
                    ci, wi = icv[k + _RING], iwv[k + _RING]
                else:
                    ci = icv_n[k + _RING - _G]
                    wi = iwv_n[k + _RING - _G]
                block_dma(ctx_tab_t, ci >> 7, rc[slot], sem_c)
                block_dma(word_tab_t, wi >> 7, rw[slot], sem_w)
            acc[pl.ds(g * _G, _G)] = res
            return 0

        lax.fori_loop(0, _NG, body, 0)

        for k in range(_RING):
            drain_one(ctx_tab_t, rc[k], sem_c)
            drain_one(word_tab_t, rw[k], sem_w)

        pltpu.sync_copy(acc, out_hbm.at[pl.ds(base, _RPW)])

    return sc_kernel


_sc_kernel = _make_sc_kernel()


def kernel(ctx, word, ctx_table, word_table):
    out = _sc_kernel(ctx.astype(jnp.int32), word.astype(jnp.int32),
                     ctx_table.T, word_table.T)
    return out.reshape(_B, 1)


# speedup vs baseline: 1.1999x; 1.1999x over previous
import functools

import jax
import jax.numpy as jnp
from jax import lax
from jax.experimental import pallas as pl
from jax.experimental.pallas import tpu as pltpu
from jax.experimental.pallas import tpu_sc as plsc

_B = 16384
_V = 1000000
_D = 64
_NC = 2
_NS = 16
_NW = _NC * _NS
_RPW = _B // _NW
_G = 16
_NG = _RPW // _G
_RC = 8
_RW = 4


def _make_sc_kernel():
    mesh = plsc.VectorSubcoreMesh(core_axis_name="c", subcore_axis_name="s")

    @functools.partial(
        pl.kernel,
        out_type=jax.ShapeDtypeStruct((_B,), jnp.float32),
        mesh=mesh,
        scratch_types=[
            pltpu.VMEM((_RPW,), jnp.int32),
            pltpu.VMEM((_RPW,), jnp.int32),
            [pltpu.VMEM((_D, 128), jnp.float32) for _ in range(_RC)],
            [pltpu.VMEM((_D, 128), jnp.float32) for _ in range(_RW)],
            pltpu.VMEM((_RPW,), jnp.float32),
            pltpu.SemaphoreType.DMA,
            pltpu.SemaphoreType.DMA,
        ],
        compiler_params=pltpu.CompilerParams(needs_layout_passes=False),
    )
    def sc_kernel(ctx_hbm, word_hbm, ctx_tab_t, word_tab_t, out_hbm,
                  ctx_idx, word_idx, rc, rw, acc, sem_c, sem_w):
        wid = lax.axis_index("s") * _NC + lax.axis_index("c")
        base = wid * _RPW

        pltpu.sync_copy(ctx_hbm.at[pl.ds(base, _RPW)], ctx_idx)
        pltpu.sync_copy(word_hbm.at[pl.ds(base, _RPW)], word_idx)

        lane = lax.iota(jnp.int32, 16)
        dvec = [lane + 16 * m for m in range(_D // 16)]

        def block_dma(tab, col_blk, dst, sem):
            start = pl.multiple_of(col_blk * 128, 128)
            pltpu.async_copy(tab.at[:, pl.ds(start, 128)], dst, sem)

        def drain_one(tab, dst, sem):
            pltpu.make_async_copy(tab.at[:, pl.ds(0, 128)], dst, sem).wait()

        def extract_dot(cbuf, wbuf, c_lane, w_lane):
            c_col = jnp.full((16,), c_lane, jnp.int32)
            w_col = jnp.full((16,), w_lane, jnp.int32)
            s = jnp.zeros((16,), jnp.float32)
            for m in range(_D // 16):
                a = plsc.load_gather(cbuf, [dvec[m], c_col])
                b = plsc.load_gather(wbuf, [dvec[m], w_col])
                s = s + a * b
            return jnp.sum(s)

        icv0 = ctx_idx[pl.ds(0, _G)]
        iwv0 = word_idx[pl.ds(0, _G)]
        for k in range(_RC):
            block_dma(ctx_tab_t, icv0[k] >> 7, rc[k], sem_c)
        for k in range(_RW):
            block_dma(word_tab_t, iwv0[k] >> 7, rw[k], sem_w)

        def body(g, _):
            icv = ctx_idx[pl.ds(g * _G, _G)]
            iwv = word_idx[pl.ds(g * _G, _G)]
            nxt = jnp.minimum(g + 1, _NG - 1)
            icv_n = ctx_idx[pl.ds(nxt * _G, _G)]
            iwv_n = word_idx[pl.ds(nxt * _G, _G)]
            res = jnp.zeros((16,), jnp.float32)
            for k in range(_G):
                sc_slot = k % _RC
                sw_slot = k % _RW
                drain_one(ctx_tab_t, rc[sc_slot], sem_c)
                drain_one(word_tab_t, rw[sw_slot], sem_w)
                d = extract_dot(rc[sc_slot], rw[sw_slot],
                                icv[k] & 127, iwv[k] & 127)
                res = jnp.where(lane == k, d, res)
                if k + _RC < _G:
                    ci = icv[k + _RC]
                else:
                    ci = icv_n[k + _RC - _G]
                if k + _RW < _G:
                    wi = iwv[k + _RW]
                else:
                    wi = iwv_n[k + _RW - _G]
                block_dma(ctx_tab_t, ci >> 7, rc[sc_slot], sem_c)
                block_dma(word_tab_t, wi >> 7, rw[sw_slot], sem_w)
            acc[pl.ds(g * _G, _G)] = res
            return 0

        lax.fori_loop(0, _NG, body, 0)

        for k in range(_RC):
            drain_one(ctx_tab_t, rc[k], sem_c)
        for k in range(_RW):
            drain_one(word_tab_t, rw[k], sem_w)

        pltpu.sync_copy(acc, out_hbm.at[pl.ds(base, _RPW)])

    return sc_kernel


_sc_kernel = _make_sc_kernel()


def kernel(ctx, word, ctx_table, word_table):
    out = _sc_kernel(ctx.astype(jnp.int32), word.astype(jnp.int32),
                     ctx_table.T, word_table.T)
    return out.reshape(_B, 1)


# speedup vs baseline: 1.2002x; 1.0002x over previous
import functools

import jax
import jax.numpy as jnp
from jax import lax
from jax.experimental import pallas as pl
from jax.experimental.pallas import tpu as pltpu
from jax.experimental.pallas import tpu_sc as plsc

_B = 16384
_V = 1000000
_D = 64
_NC = 2
_NS = 16
_NW = _NC * _NS
_RPW = _B // _NW
_G = 16
_NG = _RPW // _G
_RING = 4


def _make_sc_kernel():
    mesh = plsc.VectorSubcoreMesh(core_axis_name="c", subcore_axis_name="s")

    @functools.partial(
        pl.kernel,
        out_type=jax.ShapeDtypeStruct((_B,), jnp.float32),
        mesh=mesh,
        scratch_types=[
            pltpu.VMEM((_RPW,), jnp.int32),
            pltpu.VMEM((_RPW,), jnp.int32),
            [pltpu.VMEM((_D, 128), jnp.float32) for _ in range(_RING)],
            [pltpu.VMEM((_D, 128), jnp.float32) for _ in range(_RING)],
            pltpu.VMEM((_RPW,), jnp.float32),
            pltpu.SemaphoreType.DMA,
            pltpu.SemaphoreType.DMA,
        ],
        compiler_params=pltpu.CompilerParams(needs_layout_passes=False),
    )
    def sc_kernel(ctx_hbm, word_hbm, ctx_tab_t, word_tab_t, out_hbm,
                  ctx_idx, word_idx, rc, rw, acc, sem_c, sem_w):
        wid = lax.axis_index("s") * _NC + lax.axis_index("c")
        base = wid * _RPW

        pltpu.sync_copy(ctx_hbm.at[pl.ds(base, _RPW)], ctx_idx)
        pltpu.sync_copy(word_hbm.at[pl.ds(base, _RPW)], word_idx)

        lane = lax.iota(jnp.int32, 16)
        dvec = [lane + 16 * m for m in range(_D // 16)]

        def block_dma(tab, col_blk, dst, sem):
            start = pl.multiple_of(col_blk * 128, 128)
            pltpu.async_copy(tab.at[:, pl.ds(start, 128)], dst, sem)

        def drain_one(tab, dst, sem):
            pltpu.make_async_copy(tab.at[:, pl.ds(0, 128)], dst, sem).wait()

        def extract_dot(cbuf, wbuf, c_lane, w_lane):
            c_col = jnp.full((16,), c_lane, jnp.int32)
            w_col = jnp.full((16,), w_lane, jnp.int32)
            s = jnp.zeros((16,), jnp.float32)
            for m in range(_D // 16):
                a = plsc.load_gather(cbuf, [dvec[m], c_col])
                b = plsc.load_gather(wbuf, [dvec[m], w_col])
                s = s + a * b
            return jnp.sum(s)

        icv0 = ctx_idx[pl.ds(0, _G)]
        iwv0 = word_idx[pl.ds(0, _G)]
        for k in range(_RING):
            block_dma(ctx_tab_t, icv0[k] >> 7, rc[k], sem_c)
            block_dma(word_tab_t, iwv0[k] >> 7, rw[k], sem_w)

        def body(g, _):
            icv = ctx_idx[pl.ds(g * _G, _G)]
            iwv = word_idx[pl.ds(g * _G, _G)]
            nxt = jnp.minimum(g + 1, _NG - 1)
            icv_n = ctx_idx[pl.ds(nxt * _G, _G)]
            iwv_n = word_idx[pl.ds(nxt * _G, _G)]
            res = jnp.zeros((16,), jnp.float32)
            for k in range(_G):
                slot = k % _RING
                drain_one(ctx_tab_t, rc[slot], sem_c)
                drain_one(word_tab_t, rw[slot], sem_w)
                d = extract_dot(rc[slot], rw[slot],
                                icv[k] & 127, iwv[k] & 127)
                res = jnp.where(lane == k, d, res)
                if k + _RING < _G:
                    ci, wi = icv[k + _RING], iwv[k + _RING]
                else:
                    ci = icv_n[k + _RING - _G]
                    wi = iwv_n[k + _RING - _G]
                block_dma(ctx_tab_t, ci >> 7, rc[slot], sem_c)
                block_dma(word_tab_t, wi >> 7, rw[slot], sem_w)
            acc[pl.ds(g * _G, _G)] = res
            return 0

        lax.fori_loop(0, _NG, body, 0)

        for k in range(_RING):
            drain_one(ctx_tab_t, rc[k], sem_c)
            drain_one(word_tab_t, rw[k], sem_w)

        pltpu.sync_copy(acc, out_hbm.at[pl.ds(base, _RPW)])

    return sc_kernel


_sc_kernel = _make_sc_kernel()


def kernel(ctx, word, ctx_table, word_table):
    out = _sc_kernel(ctx.astype(jnp.int32), word.astype(jnp.int32),
                     ctx_table.T, word_table.T)
    return out.reshape(_B, 1)
